# baseline (speedup 1.0000x reference)
import jax
import jax.numpy as jnp
from jax import lax
from jax.experimental import pallas as pl
from jax.experimental.pallas import tpu as pltpu
from jax.experimental.pallas import tpu_sc as plsc

_B, _K, _PS = 8, 512, 64
_H, _W = 1024, 1024
_STRIP_ROWS = 64
_TILES = 32
_STRIPS_PER_TILE = (_B * (_H // _STRIP_ROWS)) // _TILES
_NBUF = 6


def _sc_scatter_body(logits_hbm, r_hbm, c_hbm, out_hbm,
                     strip_v, p0, p1, p2, p3, p4, p5, r_v, c_v,
                     rc_s, wl_s, s0, s1, s2, s3, s4, s5):
    bufs = (p0, p1, p2, p3, p4, p5)
    sems = (s0, s1, s2, s3, s4, s5)
    nc = 2
    wid = lax.axis_index("s") * nc + lax.axis_index("c")
    b = wid // (_TILES // _B)
    q = wid % (_TILES // _B)

    pltpu.sync_copy(r_hbm.at[b], r_v)
    pltpu.sync_copy(c_hbm.at[b], c_v)

    iota16 = lax.iota(jnp.int32, 16)
    zeros16 = jnp.zeros((16,), jnp.float32)

    def _pack(k16, carry):
        rcv = r_v[pl.ds(k16 * 16, 16)] * _W + c_v[pl.ds(k16 * 16, 16)]
        for i in range(16):
            rc_s[k16 * 16 + i] = rcv[i]
        return carry
    lax.fori_loop(0, _K // 16, _pack, 0)

    def _start(entry, buf, sem):
        k = entry >> 20
        pltpu.make_async_copy(logits_hbm.at[b, k, 0], buf, sem).start()

    def _wait(buf, sem):
        pltpu.make_async_copy(logits_hbm.at[0, 0, 0], buf, sem).wait()

    for j in range(_STRIPS_PER_TILE):
        s = q * _STRIPS_PER_TILE + j
        row0 = s * _STRIP_ROWS

        def _zrow(i, carry):
            def _zcol(g2, carry2):
                for u in range(4):
                    strip_v[i, pl.ds(g2 * 64 + u * 16, 16)] = zeros16
                return carry2
            lax.fori_loop(0, _W // 64, _zcol, 0)
            return carry
        lax.fori_loop(0, _STRIP_ROWS, _zrow, 0)

        def _scan(k, n):
            e = rc_s[k]
            r = e >> 10
            lo = jnp.maximum(r, row0)
            hi = jnp.minimum(r + _PS, row0 + _STRIP_ROWS)
            ok = hi > lo

            @pl.when(ok)
            def _():
                wl_s[n] = (k << 20) | e
            return n + ok.astype(jnp.int32)
        n = lax.fori_loop(0, _K, _scan, 0)

        for u in range(_NBUF):
            @pl.when(u < n)
            def _():
                _start(wl_s[u], bufs[u], sems[u])

        def _quad(i4, carry):
            for u in range(_NBUF):
                idx = i4 * _NBUF + u

                @pl.when(idx < n)
                def _():
                    e = wl_s[idx]
                    r = (e >> 10) & 1023
                    c = e & 1023
                    lo = jnp.maximum(r, row0)
                    hi = jnp.minimum(r + _PS, row0 + _STRIP_ROWS)
                    buf = bufs[u]
                    _wait(buf, sems[u])

                    @plsc.parallel_loop(lo, hi, unroll=2)
                    def _row(h):
                        dh = h - r
                        row = lax.broadcast(h - row0, (16,))
                        col = c + iota16
                        vs = [buf[dh, pl.ds(g * 16, 16)] for g in range(4)]
                        for g in range(4):
                            plsc.addupdate_scatter(
                                strip_v, [row, col + g * 16], vs[g])

                    nxt = idx + _NBUF

                    @pl.when(nxt < n)
                    def _():
                        _start(wl_s[nxt], buf, sems[u])
            return carry
        lax.fori_loop(0, (n + _NBUF - 1) // _NBUF, _quad, 0)

        pltpu.sync_copy(strip_v, out_hbm.at[b, pl.ds(row0, _STRIP_ROWS)])


def _sc_scatter(logits_5d, coords_r, coords_c):
    mesh = plsc.VectorSubcoreMesh(core_axis_name="c", subcore_axis_name="s")
    return pl.kernel(
        _sc_scatter_body,
        mesh=mesh,
        compiler_params=pltpu.CompilerParams(needs_layout_passes=False),
        out_type=jax.ShapeDtypeStruct((_B, _H, _W), jnp.float32),
        scratch_types=[
            pltpu.VMEM((_STRIP_ROWS, _W), jnp.float32),
            pltpu.VMEM((_PS, _PS), jnp.float32),
            pltpu.VMEM((_PS, _PS), jnp.float32),
            pltpu.VMEM((_PS, _PS), jnp.float32),
            pltpu.VMEM((_PS, _PS), jnp.float32),
            pltpu.VMEM((_PS, _PS), jnp.float32),
            pltpu.VMEM((_PS, _PS), jnp.float32),
            pltpu.VMEM((_K,), jnp.int32),
            pltpu.VMEM((_K,), jnp.int32),
            pltpu.SMEM((_K,), jnp.int32),
            pltpu.SMEM((_K,), jnp.int32),
            pltpu.SemaphoreType.DMA,
            pltpu.SemaphoreType.DMA,
            pltpu.SemaphoreType.DMA,
            pltpu.SemaphoreType.DMA,
            pltpu.SemaphoreType.DMA,
            pltpu.SemaphoreType.DMA,
        ],
    )(logits_5d, coords_r, coords_c)


def _tc_normalize_kernel(raw_ref, r_ref, c_ref, out_ref):
    raw = raw_ref[0]
    r = r_ref[0]
    c = c_ref[0]
    h_iota = lax.broadcasted_iota(jnp.int32, (_H, _K), 0)
    w_iota = lax.broadcasted_iota(jnp.int32, (_K, _W), 1)
    rmat = ((h_iota >= r) & (h_iota < r + _PS)).astype(jnp.bfloat16)
    cmat = ((w_iota >= c) & (w_iota < c + _PS)).astype(jnp.bfloat16)
    counts = jnp.dot(rmat, cmat, preferred_element_type=jnp.float32)
    covered = counts >= 0.5
    safe = jnp.maximum(counts, 1.0)
    out_ref[0] = jnp.where(covered, raw / safe, jnp.float32(-10.0))


def _tc_normalize(raw, coords_r3, coords_c3):
    return pl.pallas_call(
        _tc_normalize_kernel,
        grid=(_B,),
        in_specs=[
            pl.BlockSpec((1, _H, _W), lambda i: (i, 0, 0)),
            pl.BlockSpec((1, 1, _K), lambda i: (i, 0, 0)),
            pl.BlockSpec((1, _K, 1), lambda i: (i, 0, 0)),
        ],
        out_specs=pl.BlockSpec((1, _H, _W), lambda i: (i, 0, 0)),
        out_shape=jax.ShapeDtypeStruct((_B, _H, _W), jnp.float32),
    )(raw, coords_r3, coords_c3)


def kernel(patch_logits, coords, output_size, prev_pred):
    Bb, Kk, Cc, ph, pw = patch_logits.shape
    coords_r = coords[:, :, 0]
    coords_c = coords[:, :, 1]
    raw = _sc_scatter(patch_logits, coords_r, coords_c)
    out = _tc_normalize(raw, coords_r.reshape(Bb, 1, Kk),
                        coords_c.reshape(Bb, Kk, 1))
    return out.reshape(Bb, Cc, _H, _W)


# speedup vs baseline: 1.0627x; 1.0627x over previous
import jax
import jax.numpy as jnp
from jax import lax
from jax.experimental import pallas as pl
from jax.experimental.pallas import tpu as pltpu
from jax.experimental.pallas import tpu_sc as plsc

_B, _K, _PS = 8, 512, 64
_H, _W = 1024, 1024
_STRIP_ROWS = 64
_TILES = 32
_STRIPS_PER_TILE = (_B * (_H // _STRIP_ROWS)) // _TILES
_NBUF = 6


def _sc_scatter_body(logits_hbm, r_hbm, c_hbm, out_hbm,
                     strip_v, p0, p1, p2, p3, p4, p5, r_v, c_v,
                     rc_s, wl_s, s0, s1, s2, s3, s4, s5):
    bufs = (p0, p1, p2, p3, p4, p5)
    sems = (s0, s1, s2, s3, s4, s5)
    nc = 2
    wid = lax.axis_index("s") * nc + lax.axis_index("c")
    b = wid // (_TILES // _B)
    q = wid % (_TILES // _B)

    pltpu.sync_copy(r_hbm.at[b], r_v)
    pltpu.sync_copy(c_hbm.at[b], c_v)

    iota16 = lax.iota(jnp.int32, 16)
    zeros16 = jnp.zeros((16,), jnp.float32)

    def _pack(k16, carry):
        rcv = r_v[pl.ds(k16 * 16, 16)] * _W + c_v[pl.ds(k16 * 16, 16)]
        for i in range(16):
            rc_s[k16 * 16 + i] = rcv[i]
        return carry
    lax.fori_loop(0, _K // 16, _pack, 0)

    for j in range(_STRIPS_PER_TILE):
        s = q * _STRIPS_PER_TILE + j
        row0 = s * _STRIP_ROWS

        def _chunks(entry):
            k = entry >> 20
            r = (entry >> 10) & 1023
            lo = jnp.maximum(r, row0)
            hi = jnp.minimum(r + _PS, row0 + _STRIP_ROWS)
            s1 = lo - r
            sa = pl.multiple_of(jnp.minimum(s1 & ~7, 32), 8)
            need_b = (hi - r) > (sa + 32)
            return k, sa, need_b

        def _start(entry, buf, sem):
            k, sa, need_b = _chunks(entry)
            pltpu.make_async_copy(
                logits_hbm.at[b, k, 0, pl.ds(sa, 32)],
                buf.at[pl.ds(sa, 32)], sem).start()

            @pl.when(need_b)
            def _():
                pltpu.make_async_copy(
                    logits_hbm.at[b, k, 0, pl.ds(32, 32)],
                    buf.at[pl.ds(32, 32)], sem).start()

        def _wait(entry, buf, sem):
            _, _, need_b = _chunks(entry)
            pltpu.make_async_copy(
                logits_hbm.at[0, 0, 0, pl.ds(0, 32)],
                buf.at[pl.ds(0, 32)], sem).wait()

            @pl.when(need_b)
            def _():
                pltpu.make_async_copy(
                    logits_hbm.at[0, 0, 0, pl.ds(0, 32)],
                    buf.at[pl.ds(32, 32)], sem).wait()

        def _zrow(i, carry):
            def _zcol(g2, carry2):
                for u in range(4):
                    strip_v[i, pl.ds(g2 * 64 + u * 16, 16)] = zeros16
                return carry2
            lax.fori_loop(0, _W // 64, _zcol, 0)
            return carry
        lax.fori_loop(0, _STRIP_ROWS, _zrow, 0)

        def _scan(k, n):
            e = rc_s[k]
            r = e >> 10
            lo = jnp.maximum(r, row0)
            hi = jnp.minimum(r + _PS, row0 + _STRIP_ROWS)
            ok = hi > lo

            @pl.when(ok)
            def _():
                wl_s[n] = (k << 20) | e
            return n + ok.astype(jnp.int32)
        n = lax.fori_loop(0, _K, _scan, 0)

        for u in range(_NBUF):
            @pl.when(u < n)
            def _():
                _start(wl_s[u], bufs[u], sems[u])

        def _quad(i4, carry):
            for u in range(_NBUF):
                idx = i4 * _NBUF + u

                @pl.when(idx < n)
                def _():
                    e = wl_s[idx]
                    r = (e >> 10) & 1023
                    c = e & 1023
                    lo = jnp.maximum(r, row0)
                    hi = jnp.minimum(r + _PS, row0 + _STRIP_ROWS)
                    buf = bufs[u]
                    _wait(e, buf, sems[u])

                    @plsc.parallel_loop(lo, hi, unroll=2)
                    def _row(h):
                        dh = h - r
                        row = lax.broadcast(h - row0, (16,))
                        col = c + iota16
                        vs = [buf[dh, pl.ds(g * 16, 16)] for g in range(4)]
                        for g in range(4):
                            plsc.addupdate_scatter(
                                strip_v, [row, col + g * 16], vs[g])

                    nxt = idx + _NBUF

                    @pl.when(nxt < n)
                    def _():
                        _start(wl_s[nxt], buf, sems[u])
            return carry
        lax.fori_loop(0, (n + _NBUF - 1) // _NBUF, _quad, 0)

        pltpu.sync_copy(strip_v, out_hbm.at[b, pl.ds(row0, _STRIP_ROWS)])


def _sc_scatter(logits_5d, coords_r, coords_c):
    mesh = plsc.VectorSubcoreMesh(core_axis_name="c", subcore_axis_name="s")
    return pl.kernel(
        _sc_scatter_body,
        mesh=mesh,
        compiler_params=pltpu.CompilerParams(needs_layout_passes=False),
        out_type=jax.ShapeDtypeStruct((_B, _H, _W), jnp.float32),
        scratch_types=[
            pltpu.VMEM((_STRIP_ROWS, _W), jnp.float32),
            pltpu.VMEM((_PS, _PS), jnp.float32),
            pltpu.VMEM((_PS, _PS), jnp.float32),
            pltpu.VMEM((_PS, _PS), jnp.float32),
            pltpu.VMEM((_PS, _PS), jnp.float32),
            pltpu.VMEM((_PS, _PS), jnp.float32),
            pltpu.VMEM((_PS, _PS), jnp.float32),
            pltpu.VMEM((_K,), jnp.int32),
            pltpu.VMEM((_K,), jnp.int32),
            pltpu.SMEM((_K,), jnp.int32),
            pltpu.SMEM((_K,), jnp.int32),
            pltpu.SemaphoreType.DMA,
            pltpu.SemaphoreType.DMA,
            pltpu.SemaphoreType.DMA,
            pltpu.SemaphoreType.DMA,
            pltpu.SemaphoreType.DMA,
            pltpu.SemaphoreType.DMA,
        ],
    )(logits_5d, coords_r, coords_c)


def _tc_normalize_kernel(raw_ref, r_ref, c_ref, out_ref):
    raw = raw_ref[0]
    r = r_ref[0]
    c = c_ref[0]
    h_iota = lax.broadcasted_iota(jnp.int32, (_H, _K), 0)
    w_iota = lax.broadcasted_iota(jnp.int32, (_K, _W), 1)
    rmat = ((h_iota >= r) & (h_iota < r + _PS)).astype(jnp.bfloat16)
    cmat = ((w_iota >= c) & (w_iota < c + _PS)).astype(jnp.bfloat16)
    counts = jnp.dot(rmat, cmat, preferred_element_type=jnp.float32)
    covered = counts >= 0.5
    safe = jnp.maximum(counts, 1.0)
    out_ref[0] = jnp.where(covered, raw / safe, jnp.float32(-10.0))


def _tc_normalize(raw, coords_r3, coords_c3):
    return pl.pallas_call(
        _tc_normalize_kernel,
        grid=(_B,),
        in_specs=[
            pl.BlockSpec((1, _H, _W), lambda i: (i, 0, 0)),
            pl.BlockSpec((1, 1, _K), lambda i: (i, 0, 0)),
            pl.BlockSpec((1, _K, 1), lambda i: (i, 0, 0)),
        ],
        out_specs=pl.BlockSpec((1, _H, _W), lambda i: (i, 0, 0)),
        out_shape=jax.ShapeDtypeStruct((_B, _H, _W), jnp.float32),
    )(raw, coords_r3, coords_c3)


def kernel(patch_logits, coords, output_size, prev_pred):
    Bb, Kk, Cc, ph, pw = patch_logits.shape
    coords_r = coords[:, :, 0]
    coords_c = coords[:, :, 1]
    raw = _sc_scatter(patch_logits, coords_r, coords_c)
    out = _tc_normalize(raw, coords_r.reshape(Bb, 1, Kk),
                        coords_c.reshape(Bb, Kk, 1))
    return out.reshape(Bb, Cc, _H, _W)


# speedup vs baseline: 1.0983x; 1.0335x over previous
import jax
import jax.numpy as jnp
from jax import lax
from jax.experimental import pallas as pl
from jax.experimental.pallas import tpu as pltpu
from jax.experimental.pallas import tpu_sc as plsc

_B, _K, _PS = 8, 512, 64
_H, _W = 1024, 1024
_STRIP_ROWS = 64
_TILES = 32
_STRIPS_PER_TILE = (_B * (_H // _STRIP_ROWS)) // _TILES
_NBUF = 6


def _sc_scatter_body(logits_hbm, r_hbm, c_hbm, zeros_hbm, out_hbm,
                     strip_v, p0, p1, p2, p3, p4, p5, r_v, c_v,
                     rc_s, wl_s, s0, s1, s2, s3, s4, s5, sz):
    bufs = (p0, p1, p2, p3, p4, p5)
    sems = (s0, s1, s2, s3, s4, s5)
    nc = 2
    wid = lax.axis_index("s") * nc + lax.axis_index("c")
    b = wid // (_TILES // _B)
    q = wid % (_TILES // _B)

    pltpu.sync_copy(r_hbm.at[b], r_v)
    pltpu.sync_copy(c_hbm.at[b], c_v)

    iota16 = lax.iota(jnp.int32, 16)
    zeros16 = jnp.zeros((16,), jnp.float32)

    def _pack(k16, carry):
        rcv = r_v[pl.ds(k16 * 16, 16)] * _W + c_v[pl.ds(k16 * 16, 16)]
        for i in range(16):
            rc_s[k16 * 16 + i] = rcv[i]
        return carry
    lax.fori_loop(0, _K // 16, _pack, 0)

    for j in range(_STRIPS_PER_TILE):
        s = q * _STRIPS_PER_TILE + j
        row0 = s * _STRIP_ROWS

        def _chunks(entry):
            k = entry >> 20
            r = (entry >> 10) & 1023
            lo = jnp.maximum(r, row0)
            hi = jnp.minimum(r + _PS, row0 + _STRIP_ROWS)
            s1 = lo - r
            sa = pl.multiple_of(jnp.minimum(s1 & ~7, 32), 8)
            need_b = (hi - r) > (sa + 32)
            return k, sa, need_b

        def _start(entry, buf, sem):
            k, sa, need_b = _chunks(entry)
            pltpu.make_async_copy(
                logits_hbm.at[b, k, 0, pl.ds(sa, 32)],
                buf.at[pl.ds(sa, 32)], sem).start()

            @pl.when(need_b)
            def _():
                pltpu.make_async_copy(
                    logits_hbm.at[b, k, 0, pl.ds(32, 32)],
                    buf.at[pl.ds(32, 32)], sem).start()

        def _wait(entry, buf, sem):
            _, _, need_b = _chunks(entry)
            pltpu.make_async_copy(
                logits_hbm.at[0, 0, 0, pl.ds(0, 32)],
                buf.at[pl.ds(0, 32)], sem).wait()

            @pl.when(need_b)
            def _():
                pltpu.make_async_copy(
                    logits_hbm.at[0, 0, 0, pl.ds(0, 32)],
                    buf.at[pl.ds(32, 32)], sem).wait()

        pltpu.make_async_copy(zeros_hbm, strip_v, sz).start()

        def _scan(k, n):
            e = rc_s[k]
            r = e >> 10
            lo = jnp.maximum(r, row0)
            hi = jnp.minimum(r + _PS, row0 + _STRIP_ROWS)
            ok = hi > lo

            @pl.when(ok)
            def _():
                wl_s[n] = (k << 20) | e
            return n + ok.astype(jnp.int32)
        n = lax.fori_loop(0, _K, _scan, 0)

        for u in range(_NBUF):
            @pl.when(u < n)
            def _():
                _start(wl_s[u], bufs[u], sems[u])

        pltpu.make_async_copy(zeros_hbm, strip_v, sz).wait()

        def _quad(i4, carry):
            for u in range(_NBUF):
                idx = i4 * _NBUF + u

                @pl.when(idx < n)
                def _():
                    e = wl_s[idx]
                    r = (e >> 10) & 1023
                    c = e & 1023
                    lo = jnp.maximum(r, row0)
                    hi = jnp.minimum(r + _PS, row0 + _STRIP_ROWS)
                    buf = bufs[u]
                    cols = [c + g * 16 + iota16 for g in range(4)]
                    _wait(e, buf, sems[u])

                    @plsc.parallel_loop(lo, hi, unroll=2)
                    def _row(h):
                        dh = h - r
                        row = lax.broadcast(h - row0, (16,))
                        vs = [buf[dh, pl.ds(g * 16, 16)] for g in range(4)]
                        for g in range(4):
                            plsc.addupdate_scatter(
                                strip_v, [row, cols[g]], vs[g])

                    nxt = idx + _NBUF

                    @pl.when(nxt < n)
                    def _():
                        _start(wl_s[nxt], buf, sems[u])
            return carry
        lax.fori_loop(0, (n + _NBUF - 1) // _NBUF, _quad, 0)

        pltpu.sync_copy(strip_v, out_hbm.at[b, pl.ds(row0, _STRIP_ROWS)])


def _sc_scatter(logits_5d, coords_r, coords_c, zeros):
    mesh = plsc.VectorSubcoreMesh(core_axis_name="c", subcore_axis_name="s")
    return pl.kernel(
        _sc_scatter_body,
        mesh=mesh,
        compiler_params=pltpu.CompilerParams(needs_layout_passes=False),
        out_type=jax.ShapeDtypeStruct((_B, _H, _W), jnp.float32),
        scratch_types=[
            pltpu.VMEM((_STRIP_ROWS, _W), jnp.float32),
            pltpu.VMEM((_PS, _PS), jnp.float32),
            pltpu.VMEM((_PS, _PS), jnp.float32),
            pltpu.VMEM((_PS, _PS), jnp.float32),
            pltpu.VMEM((_PS, _PS), jnp.float32),
            pltpu.VMEM((_PS, _PS), jnp.float32),
            pltpu.VMEM((_PS, _PS), jnp.float32),
            pltpu.VMEM((_K,), jnp.int32),
            pltpu.VMEM((_K,), jnp.int32),
            pltpu.SMEM((_K,), jnp.int32),
            pltpu.SMEM((_K,), jnp.int32),
            pltpu.SemaphoreType.DMA,
            pltpu.SemaphoreType.DMA,
            pltpu.SemaphoreType.DMA,
            pltpu.SemaphoreType.DMA,
            pltpu.SemaphoreType.DMA,
            pltpu.SemaphoreType.DMA,
            pltpu.SemaphoreType.DMA,
        ],
    )(logits_5d, coords_r, coords_c, zeros)


def _tc_normalize_kernel(raw_ref, r_ref, c_ref, out_ref):
    raw = raw_ref[0]
    r = r_ref[0]
    c = c_ref[0]
    h_iota = lax.broadcasted_iota(jnp.int32, (_H, _K), 0)
    w_iota = lax.broadcasted_iota(jnp.int32, (_K, _W), 1)
    rmat = ((h_iota >= r) & (h_iota < r + _PS)).astype(jnp.bfloat16)
    cmat = ((w_iota >= c) & (w_iota < c + _PS)).astype(jnp.bfloat16)
    counts = jnp.dot(rmat, cmat, preferred_element_type=jnp.float32)
    covered = counts >= 0.5
    safe = jnp.maximum(counts, 1.0)
    out_ref[0] = jnp.where(covered, raw / safe, jnp.float32(-10.0))


def _tc_normalize(raw, coords_r3, coords_c3):
    return pl.pallas_call(
        _tc_normalize_kernel,
        grid=(_B,),
        in_specs=[
            pl.BlockSpec((1, _H, _W), lambda i: (i, 0, 0)),
            pl.BlockSpec((1, 1, _K), lambda i: (i, 0, 0)),
            pl.BlockSpec((1, _K, 1), lambda i: (i, 0, 0)),
        ],
        out_specs=pl.BlockSpec((1, _H, _W), lambda i: (i, 0, 0)),
        out_shape=jax.ShapeDtypeStruct((_B, _H, _W), jnp.float32),
    )(raw, coords_r3, coords_c3)


def kernel(patch_logits, coords, output_size, prev_pred):
    Bb, Kk, Cc, ph, pw = patch_logits.shape
    coords_r = coords[:, :, 0]
    coords_c = coords[:, :, 1]
    zeros = jnp.zeros((_STRIP_ROWS, _W), jnp.float32)
    raw = _sc_scatter(patch_logits, coords_r, coords_c, zeros)
    out = _tc_normalize(raw, coords_r.reshape(Bb, 1, Kk),
                        coords_c.reshape(Bb, Kk, 1))
    return out.reshape(Bb, Cc, _H, _W)
